# Initial kernel scaffold; baseline (speedup 1.0000x reference)
import jax
import jax.numpy as jnp
from jax.experimental import pallas as pl


def kernel(points, coords, feats, inds, W1, b1, W2, b2, W3, b3):
    raise NotImplementedError("write your pallas kernel here")


# speedup vs baseline: 19.5675x; 19.5675x over previous
import jax
import jax.numpy as jnp
from jax import lax
from jax.experimental import pallas as pl
from jax.experimental.pallas import tpu as pltpu

_B = 4
_N = 20000
_K = 1024
_DH = 256
_PAD = 20096


def _mlp1_body(f_ref, w1_ref, b1_ref, h_ref):
    f = f_ref[...]
    w = w1_ref[...]
    h = (f[:, 0:1] * w[0:1, :] + f[:, 1:2] * w[1:2, :]
         + f[:, 2:3] * w[2:3, :]) + b1_ref[...]
    h_ref[...] = jnp.maximum(h, 0.0)


def _mlp2_body(h_ref, p_ref, w2_ref, b2_ref, w3_ref, b3_ref, o_ref):
    a = h_ref[...] + p_ref[...]
    a = jnp.dot(a, w2_ref[...], preferred_element_type=jnp.float32) + b2_ref[...]
    a = jnp.maximum(a, 0.0)
    o_ref[...] = jnp.dot(a, w3_ref[...], preferred_element_type=jnp.float32) + b3_ref[...]


def _fps_body(px_ref, py_ref, pz_ref, out_ref, dist_ref):
    lanes = lax.broadcasted_iota(jnp.int32, (_B, _PAD), 1)
    valid = lanes < _N
    dist_ref[...] = jnp.where(valid, jnp.float32(1e10), jnp.float32(-1e30))
    out_ref[...] = jnp.zeros((_B, _K), jnp.int32)
    cols = lax.broadcasted_iota(jnp.int32, (_B, _K), 1)

    def body(i, carry):
        xl, yl, zl = carry
        dx = px_ref[...] - xl
        dy = py_ref[...] - yl
        dz = pz_ref[...] - zl
        d = dx * dx + dy * dy + dz * dz
        dist = jnp.minimum(dist_ref[...], d)
        dist_ref[...] = dist
        m = jnp.max(dist, axis=1, keepdims=True)
        idx = jnp.min(jnp.where(dist == m, lanes, 2 ** 30), axis=1, keepdims=True)
        sel = lanes == idx
        out_ref[...] = jnp.where(cols == i, idx, out_ref[...])
        ninf = jnp.float32(-jnp.inf)
        xl = jnp.max(jnp.where(sel, px_ref[...], ninf), axis=1, keepdims=True)
        yl = jnp.max(jnp.where(sel, py_ref[...], ninf), axis=1, keepdims=True)
        zl = jnp.max(jnp.where(sel, pz_ref[...], ninf), axis=1, keepdims=True)
        return (xl, yl, zl)

    x0 = px_ref[:, 0:1]
    y0 = py_ref[:, 0:1]
    z0 = pz_ref[:, 0:1]
    lax.fori_loop(1, _K, body, (x0, y0, z0))


def kernel(points, coords, feats, inds, W1, b1, W2, b2, W3, b3):
    R = 2000
    G = (_B * _N) // R
    h = pl.pallas_call(
        _mlp1_body,
        grid=(G,),
        in_specs=[
            pl.BlockSpec((R, 3), lambda i: (i, 0)),
            pl.BlockSpec((3, _DH), lambda i: (0, 0)),
            pl.BlockSpec((1, _DH), lambda i: (0, 0)),
        ],
        out_specs=pl.BlockSpec((R, _DH), lambda i: (i, 0)),
        out_shape=jax.ShapeDtypeStruct((_B * _N, _DH), jnp.float32),
    )(feats, W1, b1.reshape(1, _DH))

    batch_ids = coords[:, 0]
    coarse = coords[:, 1:] // 2
    keys = ((batch_ids * 64 + coarse[:, 0]) * 64 + coarse[:, 1]) * 64 + coarse[:, 2]
    _, inv = jnp.unique(keys, return_inverse=True, size=keys.shape[0], fill_value=0)
    inv = inv.reshape(-1)
    S = _B * _N
    sums = jax.ops.segment_sum(h, inv, num_segments=S)
    cnts = jax.ops.segment_sum(jnp.ones((S, 1), jnp.float32), inv, num_segments=S)
    pooled = (sums / jnp.maximum(cnts, 1.0))[inv]

    features = pl.pallas_call(
        _mlp2_body,
        grid=(G,),
        in_specs=[
            pl.BlockSpec((R, _DH), lambda i: (i, 0)),
            pl.BlockSpec((R, _DH), lambda i: (i, 0)),
            pl.BlockSpec((_DH, _DH), lambda i: (0, 0)),
            pl.BlockSpec((1, _DH), lambda i: (0, 0)),
            pl.BlockSpec((_DH, _DH), lambda i: (0, 0)),
            pl.BlockSpec((1, _DH), lambda i: (0, 0)),
        ],
        out_specs=pl.BlockSpec((R, _DH), lambda i: (i, 0)),
        out_shape=jax.ShapeDtypeStruct((_B * _N, _DH), jnp.float32),
    )(h, pooled, W2, b2.reshape(1, _DH), W3, b3.reshape(1, _DH))

    inds2 = inds.reshape(_B, _N)
    pv = jnp.take_along_axis(points, inds2[..., None], axis=1)
    pvp = jnp.pad(pv, ((0, 0), (0, _PAD - _N), (0, 0)))
    px = pvp[..., 0]
    py = pvp[..., 1]
    pz = pvp[..., 2]

    samp = pl.pallas_call(
        _fps_body,
        out_shape=jax.ShapeDtypeStruct((_B, _K), jnp.int32),
        scratch_shapes=[pltpu.VMEM((_B, _PAD), jnp.float32)],
    )(px, py, pz)

    fp2_inds = jnp.take_along_axis(inds2, samp, axis=1)
    fp2_xyz = jnp.take_along_axis(pv, samp[..., None], axis=1)
    feat4 = features.reshape(_B, _N, _DH)
    fp2_features = jnp.take_along_axis(feat4, samp[..., None], axis=1).transpose(0, 2, 1)
    return fp2_features, fp2_xyz, fp2_inds


# speedup vs baseline: 20.7501x; 1.0604x over previous
import jax
import jax.numpy as jnp
from jax import lax
from jax.experimental import pallas as pl
from jax.experimental.pallas import tpu as pltpu

_B = 4
_N = 20000
_K = 1024
_DH = 256
_PAD = 20224
_H = _PAD // 2


def _mlp1_body(f_ref, w1_ref, b1_ref, h_ref):
    f = f_ref[...]
    w = w1_ref[...]
    h = (f[:, 0:1] * w[0:1, :] + f[:, 1:2] * w[1:2, :]
         + f[:, 2:3] * w[2:3, :]) + b1_ref[...]
    h_ref[...] = jnp.maximum(h, 0.0)


def _mlp2_body(h_ref, p_ref, w2_ref, b2_ref, w3_ref, b3_ref, o_ref):
    a = h_ref[...] + p_ref[...]
    a = jnp.dot(a, w2_ref[...], preferred_element_type=jnp.float32) + b2_ref[...]
    a = jnp.maximum(a, 0.0)
    o_ref[...] = jnp.dot(a, w3_ref[...], preferred_element_type=jnp.float32) + b3_ref[...]


def _pair(x, parity, combine):
    up = jnp.concatenate([x[1:], x[:1]], axis=0)
    dn = jnp.concatenate([x[-1:], x[:-1]], axis=0)
    partner = jnp.where(parity == 0, up, dn)
    return combine(x, partner)


def _fps_body(px_ref, py_ref, pz_ref, out_ref, dist_ref):
    lanes = lax.broadcasted_iota(jnp.int32, (2 * _B, _H), 1)
    parity = lax.broadcasted_iota(jnp.int32, (2 * _B, 1), 0) % 2
    gidx = lanes + parity * _H
    valid = gidx < _N
    dist_ref[...] = jnp.where(valid, jnp.float32(1e10), jnp.float32(-1e30))
    out_ref[...] = jnp.zeros((2 * _B, _K), jnp.int32)
    cols = lax.broadcasted_iota(jnp.int32, (2 * _B, _K), 1)
    ninf = jnp.float32(-jnp.inf)

    def body(i, carry):
        xl, yl, zl = carry
        dx = px_ref[...] - xl
        dy = py_ref[...] - yl
        dz = pz_ref[...] - zl
        d = dx * dx + dy * dy + dz * dz
        dist = jnp.minimum(dist_ref[...], d)
        dist_ref[...] = dist
        m = _pair(jnp.max(dist, axis=1, keepdims=True), parity, jnp.maximum)
        idx = _pair(jnp.min(jnp.where(dist == m, gidx, 2 ** 30), axis=1,
                            keepdims=True), parity, jnp.minimum)
        sel = gidx == idx
        out_ref[...] = jnp.where(cols == i, idx, out_ref[...])
        xl = _pair(jnp.max(jnp.where(sel, px_ref[...], ninf), axis=1,
                           keepdims=True), parity, jnp.maximum)
        yl = _pair(jnp.max(jnp.where(sel, py_ref[...], ninf), axis=1,
                           keepdims=True), parity, jnp.maximum)
        zl = _pair(jnp.max(jnp.where(sel, pz_ref[...], ninf), axis=1,
                           keepdims=True), parity, jnp.maximum)
        return (xl, yl, zl)

    def bcast0(r):
        c = r[:, 0:1]
        return jnp.where(parity == 0, c, jnp.concatenate([c[-1:], c[:-1]], axis=0))

    x0 = bcast0(px_ref)
    y0 = bcast0(py_ref)
    z0 = bcast0(pz_ref)
    lax.fori_loop(1, _K, body, (x0, y0, z0))


def kernel(points, coords, feats, inds, W1, b1, W2, b2, W3, b3):
    R = 2000
    G = (_B * _N) // R
    h = pl.pallas_call(
        _mlp1_body,
        grid=(G,),
        in_specs=[
            pl.BlockSpec((R, 3), lambda i: (i, 0)),
            pl.BlockSpec((3, _DH), lambda i: (0, 0)),
            pl.BlockSpec((1, _DH), lambda i: (0, 0)),
        ],
        out_specs=pl.BlockSpec((R, _DH), lambda i: (i, 0)),
        out_shape=jax.ShapeDtypeStruct((_B * _N, _DH), jnp.float32),
    )(feats, W1, b1.reshape(1, _DH))

    batch_ids = coords[:, 0]
    coarse = coords[:, 1:] // 2
    keys = ((batch_ids * 64 + coarse[:, 0]) * 64 + coarse[:, 1]) * 64 + coarse[:, 2]
    _, inv = jnp.unique(keys, return_inverse=True, size=keys.shape[0], fill_value=0)
    inv = inv.reshape(-1)
    S = _B * _N
    sums = jax.ops.segment_sum(h, inv, num_segments=S)
    cnts = jax.ops.segment_sum(jnp.ones((S, 1), jnp.float32), inv, num_segments=S)
    pooled = (sums / jnp.maximum(cnts, 1.0))[inv]

    features = pl.pallas_call(
        _mlp2_body,
        grid=(G,),
        in_specs=[
            pl.BlockSpec((R, _DH), lambda i: (i, 0)),
            pl.BlockSpec((R, _DH), lambda i: (i, 0)),
            pl.BlockSpec((_DH, _DH), lambda i: (0, 0)),
            pl.BlockSpec((1, _DH), lambda i: (0, 0)),
            pl.BlockSpec((_DH, _DH), lambda i: (0, 0)),
            pl.BlockSpec((1, _DH), lambda i: (0, 0)),
        ],
        out_specs=pl.BlockSpec((R, _DH), lambda i: (i, 0)),
        out_shape=jax.ShapeDtypeStruct((_B * _N, _DH), jnp.float32),
    )(h, pooled, W2, b2.reshape(1, _DH), W3, b3.reshape(1, _DH))

    inds2 = inds.reshape(_B, _N)
    pv = jnp.take_along_axis(points, inds2[..., None], axis=1)
    pvp = jnp.pad(pv, ((0, 0), (0, _PAD - _N), (0, 0)))
    px = pvp[..., 0].reshape(2 * _B, _H)
    py = pvp[..., 1].reshape(2 * _B, _H)
    pz = pvp[..., 2].reshape(2 * _B, _H)

    samp8 = pl.pallas_call(
        _fps_body,
        out_shape=jax.ShapeDtypeStruct((2 * _B, _K), jnp.int32),
        scratch_shapes=[pltpu.VMEM((2 * _B, _H), jnp.float32)],
    )(px, py, pz)
    samp = samp8[::2]

    fp2_inds = jnp.take_along_axis(inds2, samp, axis=1)
    fp2_xyz = jnp.take_along_axis(pv, samp[..., None], axis=1)
    feat4 = features.reshape(_B, _N, _DH)
    fp2_features = jnp.take_along_axis(feat4, samp[..., None], axis=1).transpose(0, 2, 1)
    return fp2_features, fp2_xyz, fp2_inds


# speedup vs baseline: 22.7552x; 1.0966x over previous
import functools

import jax
import jax.numpy as jnp
from jax import lax
from jax.experimental import pallas as pl
from jax.experimental.pallas import tpu as pltpu
from jax.experimental.pallas import tpu_sc as plsc

_B = 4
_N = 20000
_T = _B * _N
_TP = 81920
_K = 1024
_DH = 256
_R = 2000
_G = _T // _R
_PAD = 20224
_H = _PAD // 2


def _mlp_head(f, w, b):
    h = f[:, 0:1] * w[0:1, :] + f[:, 1:2] * w[1:2, :] + f[:, 2:3] * w[2:3, :]
    return jnp.maximum(h + b, 0.0)


def _k1_body(fs_ref, w1_ref, b1_ref, p_ref, base_ref, carry_ref):
    g = pl.program_id(0)

    @pl.when(g == 0)
    def _init():
        carry_ref[...] = jnp.zeros_like(carry_ref)

    h = _mlp_head(fs_ref[...], w1_ref[...], b1_ref[...])
    c = h
    k = 1
    while k < _R:
        c = c + jnp.concatenate(
            [jnp.zeros((k, _DH), jnp.float32), c[:-k]], axis=0)
        k *= 2
    p_ref[...] = c
    base_ref[...] = carry_ref[...][None]
    carry_ref[...] = carry_ref[...] + jnp.broadcast_to(c[-1:, :], (8, _DH))


def _k3_body(fs_ref, ple_ref, pls_ref, be_ref, bs_ref, w_ref, cnt_ref,
             base_ref, w1_ref, b1_ref, w2_ref, b2_ref, w3_ref, b3_ref, o_ref):
    h = _mlp_head(fs_ref[...], w1_ref[...], b1_ref[...])
    bcols = lax.broadcasted_iota(jnp.int32, (1, 64), 1)
    ohe = (be_ref[...] == bcols).astype(jnp.float32)
    ohs = (bs_ref[...] == bcols).astype(jnp.float32)
    base = base_ref[...]
    pe = ple_ref[...] + jnp.dot(ohe, base, preferred_element_type=jnp.float32)
    ps = pls_ref[...] + jnp.dot(ohs, base, preferred_element_type=jnp.float32)
    pooled = (pe - w_ref[...] * ps) / cnt_ref[...]
    a = h + pooled
    a = jnp.dot(a, w2_ref[...], preferred_element_type=jnp.float32) + b2_ref[...]
    a = jnp.maximum(a, 0.0)
    o_ref[...] = jnp.dot(a, w3_ref[...], preferred_element_type=jnp.float32) + b3_ref[...]


def _pair(x, parity, combine):
    up = jnp.concatenate([x[1:], x[:1]], axis=0)
    dn = jnp.concatenate([x[-1:], x[:-1]], axis=0)
    partner = jnp.where(parity == 0, up, dn)
    return combine(x, partner)


def _fps_body(px_ref, py_ref, pz_ref, out_ref, dist_ref):
    lanes = lax.broadcasted_iota(jnp.int32, (2 * _B, _H), 1)
    parity = lax.broadcasted_iota(jnp.int32, (2 * _B, 1), 0) % 2
    gidx = lanes + parity * _H
    valid = gidx < _N
    dist_ref[...] = jnp.where(valid, jnp.float32(1e10), jnp.float32(-1e30))
    out_ref[...] = jnp.zeros((2 * _B, _K), jnp.int32)
    cols = lax.broadcasted_iota(jnp.int32, (2 * _B, _K), 1)
    ninf = jnp.float32(-jnp.inf)

    def body(i, carry):
        xl, yl, zl = carry
        dx = px_ref[...] - xl
        dy = py_ref[...] - yl
        dz = pz_ref[...] - zl
        d = dx * dx + dy * dy + dz * dz
        dist = jnp.minimum(dist_ref[...], d)
        dist_ref[...] = dist
        m = _pair(jnp.max(dist, axis=1, keepdims=True), parity, jnp.maximum)
        idx = _pair(jnp.min(jnp.where(dist == m, gidx, 2 ** 30), axis=1,
                            keepdims=True), parity, jnp.minimum)
        sel = gidx == idx
        out_ref[...] = jnp.where(cols == i, idx, out_ref[...])
        xl = _pair(jnp.max(jnp.where(sel, px_ref[...], ninf), axis=1,
                           keepdims=True), parity, jnp.maximum)
        yl = _pair(jnp.max(jnp.where(sel, py_ref[...], ninf), axis=1,
                           keepdims=True), parity, jnp.maximum)
        zl = _pair(jnp.max(jnp.where(sel, pz_ref[...], ninf), axis=1,
                           keepdims=True), parity, jnp.maximum)
        return (xl, yl, zl)

    def bcast0(r):
        c = r[:, 0:1]
        return jnp.where(parity == 0, c,
                         jnp.concatenate([c[-1:], c[:-1]], axis=0))

    lax.fori_loop(1, _K, body, (bcast0(px_ref), bcast0(py_ref), bcast0(pz_ref)))


def _sc_mesh():
    return plsc.VectorSubcoreMesh(core_axis_name="c", subcore_axis_name="s")


def _gather_rows2(table, idx_a, idx_b, ncols):
    per_w = _TP // 32
    nchunk = per_w // 128

    @functools.partial(
        pl.kernel, mesh=_sc_mesh(),
        out_type=(jax.ShapeDtypeStruct((_TP, ncols), jnp.float32),
                  jax.ShapeDtypeStruct((_TP, ncols), jnp.float32)),
        scratch_types=[pltpu.VMEM((128,), jnp.int32),
                       pltpu.VMEM((128, ncols), jnp.float32),
                       pltpu.SemaphoreType.DMA],
    )
    def k(table_hbm, ia_hbm, ib_hbm, oa_hbm, ob_hbm, idx_v, rows_v, sem):
        wid = lax.axis_index("s") * 2 + lax.axis_index("c")
        base = wid * per_w

        def chunk(off, idx_hbm, out_hbm):
            pltpu.sync_copy(idx_hbm.at[pl.ds(off, 128)], idx_v)
            pltpu.async_copy(table_hbm.at[idx_v], rows_v, sem).wait()
            pltpu.sync_copy(rows_v, out_hbm.at[pl.ds(off, 128)])

        def body(i, _):
            off = base + i * 128
            chunk(off, ia_hbm, oa_hbm)
            chunk(off, ib_hbm, ob_hbm)
            return 0

        lax.fori_loop(0, nchunk, body, 0)

    return k(table, idx_a, idx_b)


def _gather_rows1(table, idx, nrows, ncols):
    per_w = nrows // 32

    @functools.partial(
        pl.kernel, mesh=_sc_mesh(),
        out_type=jax.ShapeDtypeStruct((nrows, ncols), jnp.float32),
        scratch_types=[pltpu.VMEM((per_w,), jnp.int32),
                       pltpu.VMEM((per_w, ncols), jnp.float32),
                       pltpu.SemaphoreType.DMA],
    )
    def k(table_hbm, idx_hbm, out_hbm, idx_v, rows_v, sem):
        wid = lax.axis_index("s") * 2 + lax.axis_index("c")
        base = wid * per_w
        pltpu.sync_copy(idx_hbm.at[pl.ds(base, per_w)], idx_v)
        pltpu.async_copy(table_hbm.at[idx_v], rows_v, sem).wait()
        pltpu.sync_copy(rows_v, out_hbm.at[pl.ds(base, per_w)])

    return k(table, idx)


def _gather_feats(table, idx):
    per_w = _TP // 32
    nchunk = per_w // 128

    @functools.partial(
        pl.kernel, mesh=_sc_mesh(),
        out_type=jax.ShapeDtypeStruct((_TP, 128), jnp.float32),
        scratch_types=[pltpu.VMEM((128,), jnp.int32),
                       pltpu.VMEM((128, 128), jnp.float32),
                       pltpu.SemaphoreType.DMA],
    )
    def k(table_hbm, idx_hbm, out_hbm, idx_v, rows_v, sem):
        wid = lax.axis_index("s") * 2 + lax.axis_index("c")
        base = wid * per_w

        def body(i, _):
            off = base + i * 128
            pltpu.sync_copy(idx_hbm.at[pl.ds(off, 128)], idx_v)
            pltpu.async_copy(table_hbm.at[idx_v], rows_v, sem).wait()
            pltpu.sync_copy(rows_v, out_hbm.at[pl.ds(off, 128)])
            return 0

        lax.fori_loop(0, nchunk, body, 0)

    return k(table, idx)


def kernel(points, coords, feats, inds, W1, b1, W2, b2, W3, b3):
    i32 = jnp.int32
    f32 = jnp.float32

    keys = ((coords[:, 0] * 64 + coords[:, 1] // 2) * 64
            + coords[:, 2] // 2) * 64 + coords[:, 3] // 2
    order = jnp.argsort(keys).astype(i32)
    sk = jnp.take(keys, order)
    newflag = jnp.concatenate(
        [jnp.ones((1,), bool), sk[1:] != sk[:-1]])
    pos = jnp.arange(_T, dtype=i32)
    s = lax.cummax(jnp.where(newflag, pos, 0))
    nxt = jnp.concatenate([newflag[1:], jnp.ones((1,), bool)])
    e = lax.cummin(jnp.where(nxt, pos + 1, 2 ** 30), reverse=True)
    cnt = (e - s).astype(f32)
    pe = e - 1
    ps = jnp.maximum(s - 1, 0)
    wzero = (s > 0).astype(f32)
    be = pe // _R
    bs = ps // _R
    rank = jnp.zeros_like(order).at[order].set(pos)

    def padT(x):
        return jnp.pad(x, (0, _TP - _T))

    feats128 = jnp.pad(feats, ((0, 0), (0, 125)))
    fs16 = _gather_feats(feats128, padT(order))[:_T]

    p_local, base_blk = pl.pallas_call(
        _k1_body,
        grid=(_G,),
        in_specs=[
            pl.BlockSpec((_R, 128), lambda g: (g, 0)),
            pl.BlockSpec((3, _DH), lambda g: (0, 0)),
            pl.BlockSpec((1, _DH), lambda g: (0, 0)),
        ],
        out_specs=[
            pl.BlockSpec((_R, _DH), lambda g: (g, 0)),
            pl.BlockSpec((1, 8, _DH), lambda g: (g, 0, 0)),
        ],
        out_shape=[
            jax.ShapeDtypeStruct((_T, _DH), f32),
            jax.ShapeDtypeStruct((_G, 8, _DH), f32),
        ],
        scratch_shapes=[pltpu.VMEM((8, _DH), f32)],
    )(fs16, W1, b1.reshape(1, _DH))
    base64 = jnp.pad(base_blk[:, 0, :], ((0, 64 - _G), (0, 0)))

    ple, pls = _gather_rows2(p_local, padT(pe), padT(ps), _DH)

    features_sorted = pl.pallas_call(
        _k3_body,
        grid=(_G,),
        in_specs=[
            pl.BlockSpec((_R, 128), lambda g: (g, 0)),
            pl.BlockSpec((_R, _DH), lambda g: (g, 0)),
            pl.BlockSpec((_R, _DH), lambda g: (g, 0)),
            pl.BlockSpec((_R, 1), lambda g: (g, 0)),
            pl.BlockSpec((_R, 1), lambda g: (g, 0)),
            pl.BlockSpec((_R, 1), lambda g: (g, 0)),
            pl.BlockSpec((_R, 1), lambda g: (g, 0)),
            pl.BlockSpec((64, _DH), lambda g: (0, 0)),
            pl.BlockSpec((3, _DH), lambda g: (0, 0)),
            pl.BlockSpec((1, _DH), lambda g: (0, 0)),
            pl.BlockSpec((_DH, _DH), lambda g: (0, 0)),
            pl.BlockSpec((1, _DH), lambda g: (0, 0)),
            pl.BlockSpec((_DH, _DH), lambda g: (0, 0)),
            pl.BlockSpec((1, _DH), lambda g: (0, 0)),
        ],
        out_specs=pl.BlockSpec((_R, _DH), lambda g: (g, 0)),
        out_shape=jax.ShapeDtypeStruct((_T, _DH), f32),
    )(fs16, ple[:_T], pls[:_T],
      be.reshape(_T, 1), bs.reshape(_T, 1),
      wzero.reshape(_T, 1), cnt.reshape(_T, 1),
      base64, W1, b1.reshape(1, _DH), W2, b2.reshape(1, _DH),
      W3, b3.reshape(1, _DH))

    inds2 = inds.reshape(_B, _N)
    pv = jnp.take_along_axis(points, inds2[..., None], axis=1)
    pvp = jnp.pad(pv, ((0, 0), (0, _PAD - _N), (0, 0)))
    px = pvp[..., 0].reshape(2 * _B, _H)
    py = pvp[..., 1].reshape(2 * _B, _H)
    pz = pvp[..., 2].reshape(2 * _B, _H)
    samp8 = pl.pallas_call(
        _fps_body,
        out_shape=jax.ShapeDtypeStruct((2 * _B, _K), jnp.int32),
        scratch_shapes=[pltpu.VMEM((2 * _B, _H), f32)],
    )(px, py, pz)
    samp = samp8[::2]

    rank2 = rank.reshape(_B, _N)
    fidx = jnp.take_along_axis(rank2, samp, axis=1).reshape(_B * _K)
    frows = _gather_rows1(features_sorted, fidx, _B * _K, _DH)
    fp2_features = frows.reshape(_B, _K, _DH).transpose(0, 2, 1)
    fp2_inds = jnp.take_along_axis(inds2, samp, axis=1)
    fp2_xyz = jnp.take_along_axis(pv, samp[..., None], axis=1)
    return fp2_features, fp2_xyz, fp2_inds


# speedup vs baseline: 23.2734x; 1.0228x over previous
import functools

import jax
import jax.numpy as jnp
from jax import lax
from jax.experimental import pallas as pl
from jax.experimental.pallas import tpu as pltpu
from jax.experimental.pallas import tpu_sc as plsc

_B = 4
_N = 20000
_T = _B * _N
_TP = 81920
_K = 1024
_DH = 256
_R = 2000
_G = _T // _R
_PAD = 20224
_H = _PAD // 2


def _mlp_head(f, w, b):
    h = f[:, 0:1] * w[0:1, :] + f[:, 1:2] * w[1:2, :] + f[:, 2:3] * w[2:3, :]
    return jnp.maximum(h + b, 0.0)


def _k1_body(fs_ref, w1_ref, b1_ref, p_ref, base_ref, carry_ref):
    g = pl.program_id(0)

    @pl.when(g == 0)
    def _init():
        carry_ref[...] = jnp.zeros_like(carry_ref)

    h = _mlp_head(fs_ref[...], w1_ref[...], b1_ref[...])
    c = h
    k = 1
    while k < _R:
        c = c + jnp.concatenate(
            [jnp.zeros((k, _DH), jnp.float32), c[:-k]], axis=0)
        k *= 2
    p_ref[...] = c
    base_ref[...] = carry_ref[...][None]
    carry_ref[...] = carry_ref[...] + jnp.broadcast_to(c[-1:, :], (8, _DH))


def _k3_body(fs_ref, ple_ref, pls_ref, be_ref, bs_ref, w_ref, cnt_ref,
             base_ref, w1_ref, b1_ref, w2_ref, b2_ref, w3_ref, b3_ref, o_ref):
    h = _mlp_head(fs_ref[...], w1_ref[...], b1_ref[...])
    bcols = lax.broadcasted_iota(jnp.int32, (1, 64), 1)
    ohe = (be_ref[...] == bcols).astype(jnp.float32)
    ohs = (bs_ref[...] == bcols).astype(jnp.float32)
    base = base_ref[...]
    pe = ple_ref[...] + jnp.dot(ohe, base, preferred_element_type=jnp.float32)
    ps = pls_ref[...] + jnp.dot(ohs, base, preferred_element_type=jnp.float32)
    pooled = (pe - w_ref[...] * ps) / cnt_ref[...]
    a = h + pooled
    a = jnp.dot(a, w2_ref[...], preferred_element_type=jnp.float32) + b2_ref[...]
    a = jnp.maximum(a, 0.0)
    o_ref[...] = jnp.dot(a, w3_ref[...], preferred_element_type=jnp.float32) + b3_ref[...]


def _pair(x, parity, combine):
    up = jnp.concatenate([x[1:], x[:1]], axis=0)
    dn = jnp.concatenate([x[-1:], x[:-1]], axis=0)
    partner = jnp.where(parity == 0, up, dn)
    return combine(x, partner)


def _fps_body(px_ref, py_ref, pz_ref, out_ref, dist_ref):
    lanes = lax.broadcasted_iota(jnp.int32, (2 * _B, _H), 1)
    parity = lax.broadcasted_iota(jnp.int32, (2 * _B, 1), 0) % 2
    gidx = lanes + parity * _H
    valid = gidx < _N
    dist_ref[...] = jnp.where(valid, jnp.float32(1e10), jnp.float32(-1e30))
    out_ref[...] = jnp.zeros((2 * _B, _K), jnp.int32)
    cols = lax.broadcasted_iota(jnp.int32, (2 * _B, _K), 1)
    ninf = jnp.float32(-jnp.inf)

    def body(i, carry):
        xl, yl, zl = carry
        dx = px_ref[...] - xl
        dy = py_ref[...] - yl
        dz = pz_ref[...] - zl
        d = dx * dx + dy * dy + dz * dz
        dist = jnp.minimum(dist_ref[...], d)
        dist_ref[...] = dist
        m = _pair(jnp.max(dist, axis=1, keepdims=True), parity, jnp.maximum)
        idx = _pair(jnp.min(jnp.where(dist == m, gidx, 2 ** 30), axis=1,
                            keepdims=True), parity, jnp.minimum)
        sel = gidx == idx
        out_ref[...] = jnp.where(cols == i, idx, out_ref[...])
        xl = _pair(jnp.max(jnp.where(sel, px_ref[...], ninf), axis=1,
                           keepdims=True), parity, jnp.maximum)
        yl = _pair(jnp.max(jnp.where(sel, py_ref[...], ninf), axis=1,
                           keepdims=True), parity, jnp.maximum)
        zl = _pair(jnp.max(jnp.where(sel, pz_ref[...], ninf), axis=1,
                           keepdims=True), parity, jnp.maximum)
        return (xl, yl, zl)

    def bcast0(r):
        c = r[:, 0:1]
        return jnp.where(parity == 0, c,
                         jnp.concatenate([c[-1:], c[:-1]], axis=0))

    lax.fori_loop(1, _K, body, (bcast0(px_ref), bcast0(py_ref), bcast0(pz_ref)))


def _sc_mesh():
    return plsc.VectorSubcoreMesh(core_axis_name="c", subcore_axis_name="s")


def _gather_rows2(table, idx_a, idx_b, ncols):
    per_w = _TP // 32
    nchunk = per_w // 128

    @functools.partial(
        pl.kernel, mesh=_sc_mesh(),
        out_type=(jax.ShapeDtypeStruct((_TP, ncols), jnp.float32),
                  jax.ShapeDtypeStruct((_TP, ncols), jnp.float32)),
        scratch_types=[pltpu.VMEM((128,), jnp.int32),
                       pltpu.VMEM((128,), jnp.int32),
                       pltpu.VMEM((128, ncols), jnp.float32),
                       pltpu.VMEM((128, ncols), jnp.float32),
                       pltpu.SemaphoreType.DMA,
                       pltpu.SemaphoreType.DMA],
    )
    def k(table_hbm, ia_hbm, ib_hbm, oa_hbm, ob_hbm,
          ixa_v, ixb_v, rwa_v, rwb_v, sema, semb):
        wid = lax.axis_index("s") * 2 + lax.axis_index("c")
        base = wid * per_w

        def body(i, _):
            off = base + i * 128
            pltpu.sync_copy(ia_hbm.at[pl.ds(off, 128)], ixa_v)
            ga = pltpu.async_copy(table_hbm.at[ixa_v], rwa_v, sema)
            pltpu.sync_copy(ib_hbm.at[pl.ds(off, 128)], ixb_v)
            gb = pltpu.async_copy(table_hbm.at[ixb_v], rwb_v, semb)
            ga.wait()
            pltpu.sync_copy(rwa_v, oa_hbm.at[pl.ds(off, 128)])
            gb.wait()
            pltpu.sync_copy(rwb_v, ob_hbm.at[pl.ds(off, 128)])
            return 0

        lax.fori_loop(0, nchunk, body, 0)

    return k(table, idx_a, idx_b)


def _gather_rows1(table, idx, nrows, ncols):
    per_w = nrows // 32

    @functools.partial(
        pl.kernel, mesh=_sc_mesh(),
        out_type=jax.ShapeDtypeStruct((nrows, ncols), jnp.float32),
        scratch_types=[pltpu.VMEM((per_w,), jnp.int32),
                       pltpu.VMEM((per_w, ncols), jnp.float32),
                       pltpu.SemaphoreType.DMA],
    )
    def k(table_hbm, idx_hbm, out_hbm, idx_v, rows_v, sem):
        wid = lax.axis_index("s") * 2 + lax.axis_index("c")
        base = wid * per_w
        pltpu.sync_copy(idx_hbm.at[pl.ds(base, per_w)], idx_v)
        pltpu.async_copy(table_hbm.at[idx_v], rows_v, sem).wait()
        pltpu.sync_copy(rows_v, out_hbm.at[pl.ds(base, per_w)])

    return k(table, idx)


def _gather_feats(table, idx):
    per_w = _TP // 32
    nchunk = per_w // 128

    @functools.partial(
        pl.kernel, mesh=_sc_mesh(),
        out_type=jax.ShapeDtypeStruct((_TP, 128), jnp.float32),
        scratch_types=[pltpu.VMEM((128,), jnp.int32),
                       pltpu.VMEM((128,), jnp.int32),
                       pltpu.VMEM((128, 128), jnp.float32),
                       pltpu.VMEM((128, 128), jnp.float32),
                       pltpu.SemaphoreType.DMA,
                       pltpu.SemaphoreType.DMA],
    )
    def k(table_hbm, idx_hbm, out_hbm, ixa_v, ixb_v, rwa_v, rwb_v, sema, semb):
        wid = lax.axis_index("s") * 2 + lax.axis_index("c")
        base = wid * per_w

        def body(i, _):
            off = base + i * 256
            pltpu.sync_copy(idx_hbm.at[pl.ds(off, 128)], ixa_v)
            ga = pltpu.async_copy(table_hbm.at[ixa_v], rwa_v, sema)
            pltpu.sync_copy(idx_hbm.at[pl.ds(off + 128, 128)], ixb_v)
            gb = pltpu.async_copy(table_hbm.at[ixb_v], rwb_v, semb)
            ga.wait()
            pltpu.sync_copy(rwa_v, out_hbm.at[pl.ds(off, 128)])
            gb.wait()
            pltpu.sync_copy(rwb_v, out_hbm.at[pl.ds(off + 128, 128)])
            return 0

        lax.fori_loop(0, nchunk // 2, body, 0)

    return k(table, idx)


def kernel(points, coords, feats, inds, W1, b1, W2, b2, W3, b3):
    i32 = jnp.int32
    f32 = jnp.float32

    keys = ((coords[:, 0] * 64 + coords[:, 1] // 2) * 64
            + coords[:, 2] // 2) * 64 + coords[:, 3] // 2
    order = jnp.argsort(keys).astype(i32)
    sk = jnp.take(keys, order)
    newflag = jnp.concatenate(
        [jnp.ones((1,), bool), sk[1:] != sk[:-1]])
    pos = jnp.arange(_T, dtype=i32)
    s = lax.cummax(jnp.where(newflag, pos, 0))
    nxt = jnp.concatenate([newflag[1:], jnp.ones((1,), bool)])
    e = lax.cummin(jnp.where(nxt, pos + 1, 2 ** 30), reverse=True)
    cnt = (e - s).astype(f32)
    pe = e - 1
    ps = jnp.maximum(s - 1, 0)
    wzero = (s > 0).astype(f32)
    be = pe // _R
    bs = ps // _R
    rank = jnp.zeros_like(order).at[order].set(pos)

    def padT(x):
        return jnp.pad(x, (0, _TP - _T))

    feats128 = jnp.pad(feats, ((0, 0), (0, 125)))
    fs16 = _gather_feats(feats128, padT(order))[:_T]

    p_local, base_blk = pl.pallas_call(
        _k1_body,
        grid=(_G,),
        in_specs=[
            pl.BlockSpec((_R, 128), lambda g: (g, 0)),
            pl.BlockSpec((3, _DH), lambda g: (0, 0)),
            pl.BlockSpec((1, _DH), lambda g: (0, 0)),
        ],
        out_specs=[
            pl.BlockSpec((_R, _DH), lambda g: (g, 0)),
            pl.BlockSpec((1, 8, _DH), lambda g: (g, 0, 0)),
        ],
        out_shape=[
            jax.ShapeDtypeStruct((_T, _DH), f32),
            jax.ShapeDtypeStruct((_G, 8, _DH), f32),
        ],
        scratch_shapes=[pltpu.VMEM((8, _DH), f32)],
    )(fs16, W1, b1.reshape(1, _DH))
    base64 = jnp.pad(base_blk[:, 0, :], ((0, 64 - _G), (0, 0)))

    ple, pls = _gather_rows2(p_local, padT(pe), padT(ps), _DH)

    features_sorted = pl.pallas_call(
        _k3_body,
        grid=(_G,),
        in_specs=[
            pl.BlockSpec((_R, 128), lambda g: (g, 0)),
            pl.BlockSpec((_R, _DH), lambda g: (g, 0)),
            pl.BlockSpec((_R, _DH), lambda g: (g, 0)),
            pl.BlockSpec((_R, 1), lambda g: (g, 0)),
            pl.BlockSpec((_R, 1), lambda g: (g, 0)),
            pl.BlockSpec((_R, 1), lambda g: (g, 0)),
            pl.BlockSpec((_R, 1), lambda g: (g, 0)),
            pl.BlockSpec((64, _DH), lambda g: (0, 0)),
            pl.BlockSpec((3, _DH), lambda g: (0, 0)),
            pl.BlockSpec((1, _DH), lambda g: (0, 0)),
            pl.BlockSpec((_DH, _DH), lambda g: (0, 0)),
            pl.BlockSpec((1, _DH), lambda g: (0, 0)),
            pl.BlockSpec((_DH, _DH), lambda g: (0, 0)),
            pl.BlockSpec((1, _DH), lambda g: (0, 0)),
        ],
        out_specs=pl.BlockSpec((_R, _DH), lambda g: (g, 0)),
        out_shape=jax.ShapeDtypeStruct((_T, _DH), f32),
    )(fs16, ple[:_T], pls[:_T],
      be.reshape(_T, 1), bs.reshape(_T, 1),
      wzero.reshape(_T, 1), cnt.reshape(_T, 1),
      base64, W1, b1.reshape(1, _DH), W2, b2.reshape(1, _DH),
      W3, b3.reshape(1, _DH))

    inds2 = inds.reshape(_B, _N)
    pv = jnp.take_along_axis(points, inds2[..., None], axis=1)
    pvp = jnp.pad(pv, ((0, 0), (0, _PAD - _N), (0, 0)))
    px = pvp[..., 0].reshape(2 * _B, _H)
    py = pvp[..., 1].reshape(2 * _B, _H)
    pz = pvp[..., 2].reshape(2 * _B, _H)
    samp8 = pl.pallas_call(
        _fps_body,
        out_shape=jax.ShapeDtypeStruct((2 * _B, _K), jnp.int32),
        scratch_shapes=[pltpu.VMEM((2 * _B, _H), f32)],
    )(px, py, pz)
    samp = samp8[::2]

    rank2 = rank.reshape(_B, _N)
    fidx = jnp.take_along_axis(rank2, samp, axis=1).reshape(_B * _K)
    frows = _gather_rows1(features_sorted, fidx, _B * _K, _DH)
    fp2_features = frows.reshape(_B, _K, _DH).transpose(0, 2, 1)
    fp2_inds = jnp.take_along_axis(inds2, samp, axis=1)
    fp2_xyz = jnp.take_along_axis(pv, samp[..., None], axis=1)
    return fp2_features, fp2_xyz, fp2_inds
